# baseline (speedup 1.0000x reference)
import jax
import jax.numpy as jnp
from jax import lax
from jax.experimental import pallas as pl
from jax.experimental.pallas import tpu as pltpu
from jax.experimental.pallas import tpu_sc as plsc

_NC = 2
_NS = 16
_NW = _NC * _NS

_VOCAB = 1000000
_EMBED = 16
_BATCH = 16384
_FIELDS = 26
_B = _BATCH * _FIELDS

_CV = 1600
_NCHUNK = _VOCAB // _CV
_KMAX = -(-_NCHUNK // _NW)

_BC = _BATCH // 128
_BC_W = _BC // _NW
_TASKS = _FIELDS * _BC_W


def _prep_body(wt_hbm, scale_hbm, wc_hbm, i0, i1, q0, q1, scale_v, sems):
    ins = (i0, i1)
    outs = (q0, q1)
    sem_i = (sems[0], sems[1])
    sem_o = (sems[2], sems[3])
    wid = lax.axis_index("s") * _NC + lax.axis_index("c")
    pltpu.sync_copy(scale_hbm, scale_v)
    s = scale_v[...]
    lanes = lax.iota(jnp.int32, 16)

    def fire_in(k, buf):
        c = wid + k * _NW
        pltpu.async_copy(
            wt_hbm.at[:, pl.ds(c * _CV, _CV)], ins[buf], sem_i[buf]
        )

    def wait_in(buf):
        pltpu.make_async_copy(
            wt_hbm.at[:, pl.ds(0, _CV)], ins[buf], sem_i[buf]
        ).wait()

    def fire_out(k, buf):
        c = wid + k * _NW
        pltpu.async_copy(
            outs[buf], wc_hbm.at[pl.ds(c * _CV, _CV)], sem_o[buf]
        )

    def wait_out(buf):
        pltpu.make_async_copy(
            outs[buf], wc_hbm.at[pl.ds(0, _CV)], sem_o[buf]
        ).wait()

    @pl.when(wid < _NCHUNK)
    def _():
        fire_in(0, 0)

    def step(kh, carry):
        for b in (0, 1):
            k = 2 * kh + b
            c = wid + k * _NW

            @pl.when(c < _NCHUNK)
            def _():
                @pl.when(c + _NW < _NCHUNK)
                def _():
                    fire_in(k + 1, 1 - b)

                wait_in(b)

                @pl.when(k >= 2)
                def _():
                    wait_out(b)

                @plsc.parallel_loop(0, _CV, unroll=8)
                def _(v):
                    col = plsc.load_gather(
                        ins[b], [lanes, jnp.full((16,), v, jnp.int32)]
                    )
                    outs[b][v, :] = col * s

                fire_out(k, b)
        return carry

    lax.fori_loop(0, _KMAX // 2, step, None)
    for tail in (_KMAX - 2, _KMAX - 1):
        c = wid + tail * _NW

        @pl.when(c < _NCHUNK)
        def _():
            wait_out(tail % 2)


def _gather_body(idx_hbm, wc_hbm, out_hbm, idx_v, r0, r1, r2, r3,
                 o0, o1, o2, o3, sems):
    rows = (r0, r1, r2, r3)
    outs = (o0, o1, o2, o3)
    sem_g = (sems[0], sems[1], sems[2], sems[3])
    sem_w = (sems[4], sems[5], sems[6], sems[7])
    wid = lax.axis_index("s") * _NC + lax.axis_index("c")
    pltpu.sync_copy(idx_hbm.at[:, pl.ds(wid * _BC_W, _BC_W)], idx_v)
    lanes = lax.iota(jnp.int32, 16)

    def fire_gather(task, buf):
        f = task % _FIELDS
        bc = task // _FIELDS
        pltpu.async_copy(wc_hbm.at[idx_v.at[f, bc]], rows[buf], sem_g[buf])

    def wait_gather(buf):
        pltpu.make_async_copy(
            wc_hbm.at[idx_v.at[0, 0]], rows[buf], sem_g[buf]
        ).wait()

    def fire_write(task, buf):
        f = task % _FIELDS
        bc = task // _FIELDS
        col0 = wid * _BC_W * 128 + bc * 128
        pltpu.async_copy(
            outs[buf],
            out_hbm.at[pl.ds(f * _EMBED, _EMBED), pl.ds(col0, 128)],
            sem_w[buf],
        )

    def wait_write(buf):
        pltpu.make_async_copy(
            outs[buf],
            out_hbm.at[pl.ds(0, _EMBED), pl.ds(0, 128)],
            sem_w[buf],
        ).wait()

    fire_gather(0, 0)
    fire_gather(1, 1)

    def step(g, carry):
        for b in range(4):
            task = 4 * g + b
            wait_gather(b)

            @pl.when(task + 2 < _TASKS)
            def _():
                fire_gather(task + 2, (b + 2) % 4)

            @pl.when(task >= 4)
            def _():
                wait_write(b)

            for jp in range(8):
                ridx = jp * 16 + lanes
                for e in range(_EMBED):
                    col = plsc.load_gather(
                        rows[b], [ridx, jnp.full((16,), e, jnp.int32)]
                    )
                    outs[b][e, pl.ds(jp * 16, 16)] = col

            fire_write(task, b)
        return carry

    lax.fori_loop(0, _TASKS // 4, step, None)
    for b in range(4):
        wait_write(b)


_SC_PARAMS = pltpu.CompilerParams(
    use_tc_tiling_on_sc=False, needs_layout_passes=False
)


@jax.jit
def _run(wt, idx3, scale_vec):
    mesh = plsc.VectorSubcoreMesh(core_axis_name="c", subcore_axis_name="s")
    wc = pl.kernel(
        _prep_body,
        out_type=jax.ShapeDtypeStruct((_VOCAB, _EMBED), jnp.float32),
        mesh=mesh,
        scratch_types=[
            pltpu.VMEM((_EMBED, _CV), jnp.float32),
            pltpu.VMEM((_EMBED, _CV), jnp.float32),
            pltpu.VMEM((_CV, _EMBED), jnp.float32),
            pltpu.VMEM((_CV, _EMBED), jnp.float32),
            pltpu.VMEM((_EMBED,), jnp.float32),
            [pltpu.SemaphoreType.DMA] * 4,
        ],
        compiler_params=_SC_PARAMS,
    )(wt, scale_vec)

    out = pl.kernel(
        _gather_body,
        out_type=jax.ShapeDtypeStruct((_FIELDS * _EMBED, _BATCH), jnp.float32),
        mesh=mesh,
        scratch_types=[
            pltpu.VMEM((_FIELDS, _BC_W, 128), jnp.int32),
            pltpu.VMEM((128, _EMBED), jnp.float32),
            pltpu.VMEM((128, _EMBED), jnp.float32),
            pltpu.VMEM((128, _EMBED), jnp.float32),
            pltpu.VMEM((128, _EMBED), jnp.float32),
            pltpu.VMEM((_EMBED, 128), jnp.float32),
            pltpu.VMEM((_EMBED, 128), jnp.float32),
            pltpu.VMEM((_EMBED, 128), jnp.float32),
            pltpu.VMEM((_EMBED, 128), jnp.float32),
            [pltpu.SemaphoreType.DMA] * 8,
        ],
        compiler_params=_SC_PARAMS,
    )(idx3, wc)
    return out


def kernel(input, weight, weight_scale):
    wt = weight.T
    idx3 = input.T.astype(jnp.int32).reshape(_FIELDS, _BC, 128)
    scale_vec = jnp.broadcast_to(
        weight_scale.astype(jnp.float32), (_EMBED,)
    )
    out = _run(wt, idx3, scale_vec)
    return out.reshape(_FIELDS, _EMBED, _BATCH).transpose(2, 0, 1)


# speedup vs baseline: 2.7049x; 2.7049x over previous
import jax
import jax.numpy as jnp
from jax import lax
from jax.experimental import pallas as pl
from jax.experimental.pallas import tpu as pltpu
from jax.experimental.pallas import tpu_sc as plsc

_NC = 2
_NS = 16
_NW = _NC * _NS

_VOCAB = 1000000
_EMBED = 16
_BATCH = 16384
_FIELDS = 26

_BC = _BATCH // 128
_BC_W = _BC // _NW
_TASKS = _FIELDS * _BC_W


def _gather_body(idx_hbm, w_hbm, scale_hbm, out_hbm, idx_v, scale_v,
                 r0, r1, r2, r3, o0, o1, o2, o3, sems):
    rows = (r0, r1, r2, r3)
    outs = (o0, o1, o2, o3)
    sem_g = (sems[0], sems[1], sems[2], sems[3])
    sem_w = (sems[4], sems[5], sems[6], sems[7])
    wid = lax.axis_index("s") * _NC + lax.axis_index("c")
    pltpu.sync_copy(idx_hbm.at[:, pl.ds(wid * _BC_W, _BC_W)], idx_v)
    pltpu.sync_copy(scale_hbm, scale_v)
    s = scale_v[...]
    lanes = lax.iota(jnp.int32, 16)

    def fire_gather(task, buf):
        f = task % _FIELDS
        bc = task // _FIELDS
        pltpu.async_copy(w_hbm.at[idx_v.at[f, bc]], rows[buf], sem_g[buf])

    def wait_gather(buf):
        pltpu.make_async_copy(
            w_hbm.at[idx_v.at[0, 0]], rows[buf], sem_g[buf]
        ).wait()

    def fire_write(task, buf):
        f = task % _FIELDS
        bc = task // _FIELDS
        col0 = wid * _BC_W * 128 + bc * 128
        pltpu.async_copy(
            outs[buf],
            out_hbm.at[pl.ds(f * _EMBED, _EMBED), pl.ds(col0, 128)],
            sem_w[buf],
        )

    def wait_write(buf):
        pltpu.make_async_copy(
            outs[buf],
            out_hbm.at[pl.ds(0, _EMBED), pl.ds(0, 128)],
            sem_w[buf],
        ).wait()

    fire_gather(0, 0)
    fire_gather(1, 1)

    def step(g, carry):
        for b in range(4):
            task = 4 * g + b
            wait_gather(b)

            @pl.when(task + 2 < _TASKS)
            def _():
                fire_gather(task + 2, (b + 2) % 4)

            @pl.when(task >= 4)
            def _():
                wait_write(b)

            for jp in range(8):
                ridx = jp * 16 + lanes
                for e in range(_EMBED):
                    col = plsc.load_gather(
                        rows[b], [ridx, jnp.full((16,), e, jnp.int32)]
                    )
                    outs[b][e, pl.ds(jp * 16, 16)] = col * s

            fire_write(task, b)
        return carry

    lax.fori_loop(0, _TASKS // 4, step, None)
    for b in range(4):
        wait_write(b)


_SC_PARAMS = pltpu.CompilerParams(
    use_tc_tiling_on_sc=False, needs_layout_passes=False
)


@jax.jit
def _run(idx3, weight, scale_vec):
    mesh = plsc.VectorSubcoreMesh(core_axis_name="c", subcore_axis_name="s")
    out = pl.kernel(
        _gather_body,
        out_type=jax.ShapeDtypeStruct((_FIELDS * _EMBED, _BATCH), jnp.float32),
        mesh=mesh,
        scratch_types=[
            pltpu.VMEM((_FIELDS, _BC_W, 128), jnp.int32),
            pltpu.VMEM((_EMBED,), jnp.float32),
            pltpu.VMEM((128, _EMBED), jnp.float32),
            pltpu.VMEM((128, _EMBED), jnp.float32),
            pltpu.VMEM((128, _EMBED), jnp.float32),
            pltpu.VMEM((128, _EMBED), jnp.float32),
            pltpu.VMEM((_EMBED, 128), jnp.float32),
            pltpu.VMEM((_EMBED, 128), jnp.float32),
            pltpu.VMEM((_EMBED, 128), jnp.float32),
            pltpu.VMEM((_EMBED, 128), jnp.float32),
            [pltpu.SemaphoreType.DMA] * 8,
        ],
        compiler_params=_SC_PARAMS,
    )(idx3, weight, scale_vec)
    return out


def kernel(input, weight, weight_scale):
    idx3 = input.T.astype(jnp.int32).reshape(_FIELDS, _BC, 128)
    scale_vec = jnp.broadcast_to(
        weight_scale.astype(jnp.float32), (_EMBED,)
    )
    out = _run(idx3, weight, scale_vec)
    return out.reshape(_FIELDS, _EMBED, _BATCH).transpose(2, 0, 1)
